# baseline (speedup 1.0000x reference)
import functools

import jax
import jax.numpy as jnp
import numpy as np
from jax.experimental import pallas as pl

D_MODEL = 1024
H_TOTAL = 32
H_ACTIVE = 4
D_HEAD = D_MODEL // H_TOTAL
T_SEQ = 2048

_BT = 256


def _router_body(x_ref, wg_ref, w_ref):
    logits = jnp.dot(x_ref[...], wg_ref[...],
                     preferred_element_type=jnp.float32)
    m = jnp.max(logits, axis=-1, keepdims=True)
    e = jnp.exp(logits - m)
    probs = e / jnp.sum(e, axis=-1, keepdims=True)
    col = jax.lax.broadcasted_iota(jnp.int32, probs.shape, 1)
    p = probs
    w = jnp.zeros_like(probs)
    for _ in range(H_ACTIVE):
        mx = jnp.max(p, axis=-1, keepdims=True)
        cand = jnp.where(p == mx, col, H_TOTAL)
        first = jnp.min(cand, axis=-1, keepdims=True)
        sel = col == first
        w = jnp.where(sel, probs, w)
        p = jnp.where(sel, -jnp.inf, p)
    w_ref[...] = w


def _qkv_body(x_ref, w_ref, o_ref):
    o_ref[...] = jnp.dot(x_ref[...], w_ref[...],
                         preferred_element_type=jnp.float32)


def _attn_body(q_ref, k_ref, v_ref, g_ref, o_ref, *, scale):
    h = pl.program_id(0)
    i = pl.program_id(1)
    q = q_ref[0]
    k = k_ref[0]
    s = jax.lax.dot_general(q, k, (((1,), (1,)), ((), ())),
                            preferred_element_type=jnp.float32) * scale
    row = jax.lax.broadcasted_iota(jnp.int32, s.shape, 0) + i * _BT
    colidx = jax.lax.broadcasted_iota(jnp.int32, s.shape, 1)
    s = jnp.where(colidx <= row, s, jnp.float32(-1e9))
    m = jnp.max(s, axis=-1, keepdims=True)
    e = jnp.exp(s - m)
    attn = e / jnp.sum(e, axis=-1, keepdims=True)
    out = jnp.dot(attn, v_ref[0], preferred_element_type=jnp.float32)
    onehot = (jax.lax.broadcasted_iota(jnp.int32, (H_TOTAL, 1), 0) == h
              ).astype(jnp.float32)
    gate = jnp.dot(g_ref[...], onehot,
                   preferred_element_type=jnp.float32)
    o_ref[0] = out * gate


def _proj_body(x_ref, w_ref, o_ref):
    o_ref[...] = jnp.dot(x_ref[...], w_ref[...],
                         preferred_element_type=jnp.float32)


@jax.jit
def kernel(x, Wg, Wq, Wk, Wv, Wo):
    b, t, d = x.shape
    x2 = x.reshape(t, d)

    w = pl.pallas_call(
        _router_body,
        grid=(t // _BT,),
        in_specs=[
            pl.BlockSpec((_BT, d), lambda i: (i, 0)),
            pl.BlockSpec((d, H_TOTAL), lambda i: (0, 0)),
        ],
        out_specs=pl.BlockSpec((_BT, H_TOTAL), lambda i: (i, 0)),
        out_shape=jax.ShapeDtypeStruct((t, H_TOTAL), jnp.float32),
    )(x2, Wg)

    wqkv = jnp.concatenate([Wq, Wk, Wv], axis=1)
    bn = 512
    qkv = pl.pallas_call(
        _qkv_body,
        grid=(3 * d // bn,),
        in_specs=[
            pl.BlockSpec((t, d), lambda j: (0, 0)),
            pl.BlockSpec((d, bn), lambda j: (0, j)),
        ],
        out_specs=pl.BlockSpec((t, bn), lambda j: (0, j)),
        out_shape=jax.ShapeDtypeStruct((t, 3 * d), jnp.float32),
    )(x2, wqkv)

    qkv3 = qkv.reshape(t, 3 * H_TOTAL, D_HEAD).transpose(1, 0, 2)

    scale = 1.0 / np.sqrt(D_HEAD)
    attn_out = pl.pallas_call(
        functools.partial(_attn_body, scale=scale),
        grid=(H_TOTAL, t // _BT),
        in_specs=[
            pl.BlockSpec((1, _BT, D_HEAD), lambda h, i: (h, i, 0)),
            pl.BlockSpec((1, t, D_HEAD), lambda h, i: (H_TOTAL + h, 0, 0)),
            pl.BlockSpec((1, t, D_HEAD), lambda h, i: (2 * H_TOTAL + h, 0, 0)),
            pl.BlockSpec((_BT, H_TOTAL), lambda h, i: (i, 0)),
        ],
        out_specs=pl.BlockSpec((1, _BT, D_HEAD), lambda h, i: (h, i, 0)),
        out_shape=jax.ShapeDtypeStruct((H_TOTAL, t, D_HEAD), jnp.float32),
    )(qkv3, qkv3, qkv3, w)

    attn_flat = attn_out.transpose(1, 0, 2).reshape(t, d)

    y = pl.pallas_call(
        _proj_body,
        grid=(d // bn,),
        in_specs=[
            pl.BlockSpec((t, d), lambda j: (0, 0)),
            pl.BlockSpec((d, bn), lambda j: (0, j)),
        ],
        out_specs=pl.BlockSpec((t, bn), lambda j: (0, j)),
        out_shape=jax.ShapeDtypeStruct((t, d), jnp.float32),
    )(attn_flat, Wo)

    return y.reshape(b, t, d)
